# baseline (speedup 1.0000x reference)
import jax
import jax.numpy as jnp
from jax import lax
from jax.experimental import pallas as pl
from jax.experimental.pallas import tpu as pltpu
from jax.experimental.pallas import tpu_sc as plsc

_INFO = plsc.get_sparse_core_info()
_NC, _NS, _L = _INFO.num_cores, _INFO.num_subcores, _INFO.num_lanes
_NW = _NC * _NS

_B = 16384
_D = 128
_V = 1001
_VP = 1024

_B_SC = 8192
_B_TC = _B - _B_SC
_B_PER_W = _B_SC // _NW
_CHUNK = 128
_NCHUNK = _B_PER_W // _CHUNK

_TC_BM = 512
_ROWS_PER_SUB = 64


def _sc_body(labels_hbm, table_hbm, out_ref, idx_v, rows_v, tab_s, isem, gsems, ssems):
    cid = lax.axis_index("c")
    sid = lax.axis_index("s")
    wid = sid * _NC + cid
    base = wid * _B_PER_W

    icopy = pltpu.async_copy(
        labels_hbm.at[pl.ds(wid * _NCHUNK, _NCHUNK)], idx_v, isem
    )

    start = pl.multiple_of(sid * _ROWS_PER_SUB, _ROWS_PER_SUB)

    @pl.when(sid < _NS - 1)
    def _():
        pltpu.sync_copy(
            table_hbm.at[pl.ds(start, _ROWS_PER_SUB)],
            tab_s.at[pl.ds(start, _ROWS_PER_SUB)],
        )

    @pl.when(sid == _NS - 1)
    def _():
        tail = (_NS - 1) * _ROWS_PER_SUB
        pltpu.sync_copy(
            table_hbm.at[pl.ds(tail, _V - tail)],
            tab_s.at[pl.ds(tail, _V - tail)],
        )

    plsc.subcore_barrier()

    icopy.wait()
    gathers = [
        pltpu.async_copy(
            tab_s.at[idx_v.at[j]],
            rows_v.at[pl.ds(j * _CHUNK, _CHUNK)],
            gsems.at[j],
        )
        for j in range(_NCHUNK)
    ]
    stores = []
    for j in range(_NCHUNK):
        gathers[j].wait()
        stores.append(
            pltpu.async_copy(
                rows_v.at[pl.ds(j * _CHUNK, _CHUNK)],
                out_ref.at[pl.ds(base + j * _CHUNK, _CHUNK)],
                ssems.at[j],
            )
        )
    for s in stores:
        s.wait()


def _tc_block(lab_ref, tab_ref, out_ref):
    lab = lab_ref[0, 0, :]
    onehot = (
        lab[:, None] == lax.broadcasted_iota(jnp.int32, (_TC_BM, _VP), 1)
    ).astype(jnp.float32)
    out_ref[...] = jnp.dot(
        onehot, tab_ref[...], preferred_element_type=jnp.float32
    )


@jax.jit
def _embed(labels_sc, labels_tc, table):
    tab_p = jnp.zeros((_VP, _D), jnp.float32).at[:_V].set(table)
    n_tc_blocks = _B_TC // _TC_BM
    tc_out = pl.pallas_call(
        _tc_block,
        grid=(n_tc_blocks,),
        in_specs=[
            pl.BlockSpec((1, 1, _TC_BM), lambda i: (i, 0, 0)),
            pl.BlockSpec((_VP, _D), lambda i: (0, 0)),
        ],
        out_specs=pl.BlockSpec(
            (_TC_BM, _D), lambda i: (i + _B_SC // _TC_BM, 0)
        ),
        out_shape=jax.ShapeDtypeStruct((_B, _D), jnp.float32),
    )(labels_tc.reshape(n_tc_blocks, 1, _TC_BM), tab_p)

    out = jax.new_ref(tc_out)
    mesh = plsc.VectorSubcoreMesh(core_axis_name="c", subcore_axis_name="s")
    run = pl.kernel(
        _sc_body,
        mesh=mesh,
        scratch_types=[
            pltpu.VMEM((_NCHUNK, _CHUNK), jnp.int32),
            pltpu.VMEM((_B_PER_W, _D), jnp.float32),
            pltpu.VMEM_SHARED((_V, _D), jnp.float32),
            pltpu.SemaphoreType.DMA,
            pltpu.SemaphoreType.DMA((_NCHUNK,)),
            pltpu.SemaphoreType.DMA((_NCHUNK,)),
        ],
    )
    run(labels_sc, table, out)
    return out[...]


def kernel(labels, train, table):
    del train
    labels = labels.astype(jnp.int32)
    labels_sc = labels[:_B_SC].reshape(_B_SC // _CHUNK, _CHUNK)
    labels_tc = labels[_B_SC:]
    return _embed(labels_sc, labels_tc, table)


# speedup vs baseline: 1.5105x; 1.5105x over previous
import jax
import jax.numpy as jnp
from jax import lax
from jax.experimental import pallas as pl
from jax.experimental.pallas import tpu as pltpu
from jax.experimental.pallas import tpu_sc as plsc

_INFO = plsc.get_sparse_core_info()
_NC, _NS, _L = _INFO.num_cores, _INFO.num_subcores, _INFO.num_lanes
_NW = _NC * _NS

_B = 16384
_D = 128
_V = 1001
_B_PER_W = _B // _NW
_CHUNK = 64
_NCHUNK = _B_PER_W // _CHUNK

_ROWS_PER_SUB = 64


def _gather_body(labels_hbm, table_hbm, out_hbm, idx_v, rows_v, tab_s, isem, gsems, ssems):
    cid = lax.axis_index("c")
    sid = lax.axis_index("s")
    wid = sid * _NC + cid
    base = wid * _B_PER_W

    icopy = pltpu.async_copy(
        labels_hbm.at[pl.ds(wid * _NCHUNK, _NCHUNK)], idx_v, isem
    )

    start = pl.multiple_of(sid * _ROWS_PER_SUB, _ROWS_PER_SUB)

    @pl.when(sid < _NS - 1)
    def _():
        pltpu.sync_copy(
            table_hbm.at[pl.ds(start, _ROWS_PER_SUB)],
            tab_s.at[pl.ds(start, _ROWS_PER_SUB)],
        )

    @pl.when(sid == _NS - 1)
    def _():
        tail = (_NS - 1) * _ROWS_PER_SUB
        pltpu.sync_copy(
            table_hbm.at[pl.ds(tail, _V - tail)],
            tab_s.at[pl.ds(tail, _V - tail)],
        )

    plsc.subcore_barrier()

    icopy.wait()
    gathers = [
        pltpu.async_copy(
            tab_s.at[idx_v.at[j]],
            rows_v.at[pl.ds(j * _CHUNK, _CHUNK)],
            gsems.at[j],
        )
        for j in range(_NCHUNK)
    ]
    stores = []
    for j in range(_NCHUNK):
        gathers[j].wait()
        stores.append(
            pltpu.async_copy(
                rows_v.at[pl.ds(j * _CHUNK, _CHUNK)],
                out_hbm.at[pl.ds(base + j * _CHUNK, _CHUNK)],
                ssems.at[j],
            )
        )
    for s in stores:
        s.wait()


@jax.jit
def _embed(labels2d, table):
    mesh = plsc.VectorSubcoreMesh(core_axis_name="c", subcore_axis_name="s")
    run = pl.kernel(
        _gather_body,
        out_type=jax.ShapeDtypeStruct((_B, _D), jnp.float32),
        mesh=mesh,
        scratch_types=[
            pltpu.VMEM((_NCHUNK, _CHUNK), jnp.int32),
            pltpu.VMEM((_B_PER_W, _D), jnp.float32),
            pltpu.VMEM_SHARED((_V, _D), jnp.float32),
            pltpu.SemaphoreType.DMA,
            pltpu.SemaphoreType.DMA((_NCHUNK,)),
            pltpu.SemaphoreType.DMA((_NCHUNK,)),
        ],
    )
    return run(labels2d, table)


def kernel(labels, train, table):
    del train
    labels2d = labels.astype(jnp.int32).reshape(_B // _CHUNK, _CHUNK)
    return _embed(labels2d, table)
